# baseline (speedup 1.0000x reference)
import functools

import jax
import jax.numpy as jnp
from jax import lax
from jax.experimental import pallas as pl
from jax.experimental.pallas import tpu as pltpu
from jax.experimental.pallas import tpu_sc as plsc


def _make_pair_gather(V2, D2, B):
    info = plsc.get_sparse_core_info()
    NC, NS = info.num_cores, info.num_subcores
    NW = NC * NS
    assert B % (8 * NW) == 0
    b_per_w = B // NW
    n_chunks = b_per_w // 128
    mesh = plsc.VectorSubcoreMesh(core_axis_name="c", subcore_axis_name="s")

    @functools.partial(
        pl.kernel,
        mesh=mesh,
        out_type=jax.ShapeDtypeStruct((B, D2), jnp.float32),
        scratch_types=[
            pltpu.VMEM((n_chunks, 128), jnp.int32),
            pltpu.VMEM((b_per_w, D2), jnp.float32),
            pltpu.SemaphoreType.DMA,
        ],
        compiler_params=pltpu.CompilerParams(skip_device_barrier=True),
    )
    def gather_kernel(table_hbm, idx_hbm, out_hbm, idx_v, rows_v, sem):
        wid = lax.axis_index("s") * NC + lax.axis_index("c")
        base = wid * b_per_w
        pltpu.sync_copy(
            idx_hbm.at[pl.ds(wid * n_chunks, n_chunks), :], idx_v
        )
        copies = []
        for j in range(n_chunks):
            copies.append(
                pltpu.make_async_copy(
                    table_hbm.at[idx_v.at[j]],
                    rows_v.at[pl.ds(j * 128, 128), :],
                    sem,
                )
            )
            copies[-1].start()
        for c in copies:
            c.wait()
        pltpu.sync_copy(rows_v, out_hbm.at[pl.ds(base, b_per_w), :])

    return gather_kernel


def kernel(entities, entity_table, relation_table):
    B = entities.shape[0]
    V, D = entity_table.shape
    table2 = entity_table.reshape(V // 2, 2 * D)
    idx = entities.astype(jnp.int32)
    idx2 = (idx >> 1).reshape(B // 128, 128)
    gather = _make_pair_gather(V // 2, 2 * D, B)
    pairs = gather(table2, idx2)
    odd = (idx & 1).astype(bool)
    return jnp.where(odd[:, None], pairs[:, D:], pairs[:, :D])


# speedup vs baseline: 1.7460x; 1.7460x over previous
import functools

import jax
import jax.numpy as jnp
from jax import lax
from jax.experimental import pallas as pl
from jax.experimental.pallas import tpu as pltpu
from jax.experimental.pallas import tpu_sc as plsc

_BN = 2048


def _transpose_body(lo_ref, hi_ref, out_ref):
    d = lo_ref.shape[0]
    out_ref[:, :d] = lo_ref[...].T
    out_ref[:, d:] = hi_ref[...].T


def _make_transpose(D, V):
    nblk = (V + 2 * _BN - 1) // (2 * _BN)
    n_in_blocks = (V + _BN - 1) // _BN

    return pl.pallas_call(
        _transpose_body,
        grid=(nblk,),
        in_specs=[
            pl.BlockSpec((D, _BN), lambda b: (0, 2 * b)),
            pl.BlockSpec(
                (D, _BN),
                lambda b, m=n_in_blocks - 1: (0, jnp.minimum(2 * b + 1, m)),
            ),
        ],
        out_specs=pl.BlockSpec((_BN, 2 * D), lambda b: (b, 0)),
        out_shape=jax.ShapeDtypeStruct((nblk * _BN, 2 * D), jnp.float32),
    )


def _make_pair_gather(Vp, D2, B):
    info = plsc.get_sparse_core_info()
    NC, NS = info.num_cores, info.num_subcores
    NW = NC * NS
    assert B % (8 * NW) == 0
    b_per_w = B // NW
    n_chunks = b_per_w // 128
    mesh = plsc.VectorSubcoreMesh(core_axis_name="c", subcore_axis_name="s")

    @functools.partial(
        pl.kernel,
        mesh=mesh,
        out_type=jax.ShapeDtypeStruct((B, D2), jnp.float32),
        scratch_types=[
            pltpu.VMEM((n_chunks, 128), jnp.int32),
            pltpu.VMEM((b_per_w, D2), jnp.float32),
            pltpu.SemaphoreType.DMA,
        ],
    )
    def gather_kernel(table_hbm, idx_hbm, out_hbm, idx_v, rows_v, sem):
        wid = lax.axis_index("s") * NC + lax.axis_index("c")
        base = wid * b_per_w
        pltpu.sync_copy(
            idx_hbm.at[pl.ds(wid * n_chunks, n_chunks), :], idx_v
        )
        copies = []
        for j in range(n_chunks):
            copies.append(
                pltpu.make_async_copy(
                    table_hbm.at[idx_v.at[j]],
                    rows_v.at[pl.ds(j * 128, 128), :],
                    sem,
                )
            )
            copies[-1].start()
        for c in copies:
            c.wait()
        pltpu.sync_copy(rows_v, out_hbm.at[pl.ds(base, b_per_w), :])

    return gather_kernel


def kernel(entities, entity_table, relation_table):
    B = entities.shape[0]
    V, D = entity_table.shape
    tt = entity_table.T
    table2 = _make_transpose(D, V)(tt, tt)
    idx = entities.astype(jnp.int32)
    hi = (idx >> 11) & 1
    k = ((idx >> 12) << 11) | (idx & (_BN - 1))
    gather = _make_pair_gather(table2.shape[0], 2 * D, B)
    pairs = gather(table2, k.reshape(B // 128, 128))
    return jnp.where((hi == 1)[:, None], pairs[:, D:], pairs[:, :D])


# speedup vs baseline: 2.1500x; 1.2314x over previous
import functools

import jax
import jax.numpy as jnp
from jax import lax
from jax.experimental import pallas as pl
from jax.experimental.pallas import tpu as pltpu
from jax.experimental.pallas import tpu_sc as plsc

_BN = 4096


def _transpose_body(lo_ref, hi_ref, out_ref):
    d = lo_ref.shape[0]
    out_ref[:, :d] = lo_ref[...].T
    out_ref[:, d:] = hi_ref[...].T


def _make_transpose(D, V):
    nblk = (V + 2 * _BN - 1) // (2 * _BN)
    n_in_blocks = (V + _BN - 1) // _BN

    return pl.pallas_call(
        _transpose_body,
        grid=(nblk,),
        in_specs=[
            pl.BlockSpec((D, _BN), lambda b: (0, 2 * b)),
            pl.BlockSpec(
                (D, _BN),
                lambda b, m=n_in_blocks - 1: (0, jnp.minimum(2 * b + 1, m)),
            ),
        ],
        out_specs=pl.BlockSpec((_BN, 2 * D), lambda b: (b, 0)),
        out_shape=jax.ShapeDtypeStruct((nblk * _BN, 2 * D), jnp.float32),
    )


def _make_pair_gather(Vp, D2, B):
    info = plsc.get_sparse_core_info()
    NC, NS = info.num_cores, info.num_subcores
    NW = NC * NS
    assert B % (8 * NW) == 0
    b_per_w = B // NW
    n_chunks = b_per_w // 128
    mesh = plsc.VectorSubcoreMesh(core_axis_name="c", subcore_axis_name="s")

    @functools.partial(
        pl.kernel,
        mesh=mesh,
        out_type=jax.ShapeDtypeStruct((B, D2), jnp.float32),
        scratch_types=[
            pltpu.VMEM((n_chunks, 128), jnp.int32),
            pltpu.VMEM((b_per_w, D2), jnp.float32),
            pltpu.SemaphoreType.DMA,
        ],
    )
    def gather_kernel(table_hbm, idx_hbm, out_hbm, idx_v, rows_v, sem):
        wid = lax.axis_index("s") * NC + lax.axis_index("c")
        base = wid * b_per_w
        pltpu.sync_copy(
            idx_hbm.at[pl.ds(wid * n_chunks, n_chunks), :], idx_v
        )
        copies = []
        for j in range(n_chunks):
            copies.append(
                pltpu.make_async_copy(
                    table_hbm.at[idx_v.at[j]],
                    rows_v.at[pl.ds(j * 128, 128), :],
                    sem,
                )
            )
            copies[-1].start()
        for c in copies:
            c.wait()
        pltpu.sync_copy(rows_v, out_hbm.at[pl.ds(base, b_per_w), :])

    return gather_kernel


def kernel(entities, entity_table, relation_table):
    B = entities.shape[0]
    V, D = entity_table.shape
    tt = entity_table.T
    table2 = _make_transpose(D, V)(tt, tt)
    idx = entities.astype(jnp.int32)
    lb = _BN.bit_length() - 1
    hi = (idx >> lb) & 1
    k = ((idx >> (lb + 1)) << lb) | (idx & (_BN - 1))
    gather = _make_pair_gather(table2.shape[0], 2 * D, B)
    pairs = gather(table2, k.reshape(B // 128, 128))
    return jnp.where((hi == 1)[:, None], pairs[:, D:], pairs[:, :D])


# speedup vs baseline: 2.4083x; 1.1201x over previous
import functools

import jax
import jax.numpy as jnp
from jax import lax
from jax.experimental import pallas as pl
from jax.experimental.pallas import tpu as pltpu
from jax.experimental.pallas import tpu_sc as plsc

_BN = 8192


def _transpose_body(lo_ref, hi_ref, out_ref):
    d = lo_ref.shape[0]
    out_ref[:, :d] = lo_ref[...].T
    out_ref[:, d:] = hi_ref[...].T


def _make_transpose(D, V):
    nblk = (V + 2 * _BN - 1) // (2 * _BN)
    n_in_blocks = (V + _BN - 1) // _BN

    return pl.pallas_call(
        _transpose_body,
        grid=(nblk,),
        in_specs=[
            pl.BlockSpec((D, _BN), lambda b: (0, 2 * b)),
            pl.BlockSpec(
                (D, _BN),
                lambda b, m=n_in_blocks - 1: (0, jnp.minimum(2 * b + 1, m)),
            ),
        ],
        out_specs=pl.BlockSpec((_BN, 2 * D), lambda b: (b, 0)),
        out_shape=jax.ShapeDtypeStruct((nblk * _BN, 2 * D), jnp.float32),
    )


def _make_pair_gather(Vp, D2, B):
    info = plsc.get_sparse_core_info()
    NC, NS = info.num_cores, info.num_subcores
    NW = NC * NS
    assert B % (8 * NW) == 0
    b_per_w = B // NW
    n_chunks = b_per_w // 128
    mesh = plsc.VectorSubcoreMesh(core_axis_name="c", subcore_axis_name="s")

    @functools.partial(
        pl.kernel,
        mesh=mesh,
        out_type=jax.ShapeDtypeStruct((B, D2), jnp.float32),
        scratch_types=[
            pltpu.VMEM((n_chunks, 128), jnp.int32),
            pltpu.VMEM((b_per_w, D2), jnp.float32),
            pltpu.SemaphoreType.DMA,
        ],
    )
    def gather_kernel(table_hbm, idx_hbm, out_hbm, idx_v, rows_v, sem):
        wid = lax.axis_index("s") * NC + lax.axis_index("c")
        base = wid * b_per_w
        pltpu.sync_copy(
            idx_hbm.at[pl.ds(wid * n_chunks, n_chunks), :], idx_v
        )
        copies = []
        for j in range(n_chunks):
            copies.append(
                pltpu.make_async_copy(
                    table_hbm.at[idx_v.at[j]],
                    rows_v.at[pl.ds(j * 128, 128), :],
                    sem,
                )
            )
            copies[-1].start()
        for c in copies:
            c.wait()
        pltpu.sync_copy(rows_v, out_hbm.at[pl.ds(base, b_per_w), :])

    return gather_kernel


def kernel(entities, entity_table, relation_table):
    B = entities.shape[0]
    V, D = entity_table.shape
    tt = entity_table.T
    table2 = _make_transpose(D, V)(tt, tt)
    idx = entities.astype(jnp.int32)
    lb = _BN.bit_length() - 1
    hi = (idx >> lb) & 1
    k = ((idx >> (lb + 1)) << lb) | (idx & (_BN - 1))
    gather = _make_pair_gather(table2.shape[0], 2 * D, B)
    pairs = gather(table2, k.reshape(B // 128, 128))
    return jnp.where((hi == 1)[:, None], pairs[:, D:], pairs[:, :D])


# speedup vs baseline: 2.5424x; 1.0557x over previous
import functools

import jax
import jax.numpy as jnp
from jax import lax
from jax.experimental import pallas as pl
from jax.experimental.pallas import tpu as pltpu
from jax.experimental.pallas import tpu_sc as plsc

_BN = 16384


def _transpose_body(lo_ref, hi_ref, out_ref):
    d = lo_ref.shape[0]
    out_ref[:, :d] = lo_ref[...].T
    out_ref[:, d:] = hi_ref[...].T


def _make_transpose(D, V):
    nblk = (V + 2 * _BN - 1) // (2 * _BN)
    n_in_blocks = (V + _BN - 1) // _BN

    return pl.pallas_call(
        _transpose_body,
        grid=(nblk,),
        in_specs=[
            pl.BlockSpec((D, _BN), lambda b: (0, 2 * b)),
            pl.BlockSpec(
                (D, _BN),
                lambda b, m=n_in_blocks - 1: (0, jnp.minimum(2 * b + 1, m)),
            ),
        ],
        out_specs=pl.BlockSpec((_BN, 2 * D), lambda b: (b, 0)),
        out_shape=jax.ShapeDtypeStruct((nblk * _BN, 2 * D), jnp.float32),
    )


def _make_pair_gather(Vp, D2, B):
    info = plsc.get_sparse_core_info()
    NC, NS = info.num_cores, info.num_subcores
    NW = NC * NS
    assert B % (8 * NW) == 0
    b_per_w = B // NW
    n_chunks = b_per_w // 128
    mesh = plsc.VectorSubcoreMesh(core_axis_name="c", subcore_axis_name="s")

    @functools.partial(
        pl.kernel,
        mesh=mesh,
        out_type=jax.ShapeDtypeStruct((B, D2), jnp.float32),
        scratch_types=[
            pltpu.VMEM((n_chunks, 128), jnp.int32),
            pltpu.VMEM((b_per_w, D2), jnp.float32),
            pltpu.SemaphoreType.DMA,
        ],
    )
    def gather_kernel(table_hbm, idx_hbm, out_hbm, idx_v, rows_v, sem):
        wid = lax.axis_index("s") * NC + lax.axis_index("c")
        base = wid * b_per_w
        pltpu.sync_copy(
            idx_hbm.at[pl.ds(wid * n_chunks, n_chunks), :], idx_v
        )
        copies = []
        for j in range(n_chunks):
            copies.append(
                pltpu.make_async_copy(
                    table_hbm.at[idx_v.at[j]],
                    rows_v.at[pl.ds(j * 128, 128), :],
                    sem,
                )
            )
            copies[-1].start()
        for c in copies:
            c.wait()
        pltpu.sync_copy(rows_v, out_hbm.at[pl.ds(base, b_per_w), :])

    return gather_kernel


def kernel(entities, entity_table, relation_table):
    B = entities.shape[0]
    V, D = entity_table.shape
    tt = entity_table.T
    table2 = _make_transpose(D, V)(tt, tt)
    idx = entities.astype(jnp.int32)
    lb = _BN.bit_length() - 1
    hi = (idx >> lb) & 1
    k = ((idx >> (lb + 1)) << lb) | (idx & (_BN - 1))
    gather = _make_pair_gather(table2.shape[0], 2 * D, B)
    pairs = gather(table2, k.reshape(B // 128, 128))
    return jnp.where((hi == 1)[:, None], pairs[:, D:], pairs[:, :D])


# speedup vs baseline: 2.9506x; 1.1605x over previous
import functools

import jax
import jax.numpy as jnp
from jax import lax
from jax.experimental import pallas as pl
from jax.experimental.pallas import tpu as pltpu
from jax.experimental.pallas import tpu_sc as plsc

_BN = 16384


def _transpose_body(lo_ref, hi_ref, out_ref):
    d = lo_ref.shape[0]
    r = lax.broadcasted_iota(jnp.int32, (d, 2 * d), 0)
    c = lax.broadcasted_iota(jnp.int32, (d, 2 * d), 1)
    e_lo = (c == r).astype(jnp.float32)
    e_hi = (c == r + d).astype(jnp.float32)
    dn = (((0,), (0,)), ((), ()))
    out_ref[...] = lax.dot_general(
        lo_ref[...], e_lo, dn, preferred_element_type=jnp.float32
    ) + lax.dot_general(
        hi_ref[...], e_hi, dn, preferred_element_type=jnp.float32
    )


def _make_transpose(D, V):
    nblk = (V + 2 * _BN - 1) // (2 * _BN)
    n_in_blocks = (V + _BN - 1) // _BN

    return pl.pallas_call(
        _transpose_body,
        grid=(nblk,),
        in_specs=[
            pl.BlockSpec((D, _BN), lambda b: (0, 2 * b)),
            pl.BlockSpec(
                (D, _BN),
                lambda b, m=n_in_blocks - 1: (0, jnp.minimum(2 * b + 1, m)),
            ),
        ],
        out_specs=pl.BlockSpec((_BN, 2 * D), lambda b: (b, 0)),
        out_shape=jax.ShapeDtypeStruct((nblk * _BN, 2 * D), jnp.float32),
    )


def _make_pair_gather(Vp, D2, B):
    info = plsc.get_sparse_core_info()
    NC, NS = info.num_cores, info.num_subcores
    NW = NC * NS
    assert B % (8 * NW) == 0
    b_per_w = B // NW
    n_chunks = b_per_w // 128
    mesh = plsc.VectorSubcoreMesh(core_axis_name="c", subcore_axis_name="s")

    @functools.partial(
        pl.kernel,
        mesh=mesh,
        out_type=jax.ShapeDtypeStruct((B, D2), jnp.float32),
        scratch_types=[
            pltpu.VMEM((n_chunks, 128), jnp.int32),
            pltpu.VMEM((b_per_w, D2), jnp.float32),
            pltpu.SemaphoreType.DMA,
        ],
    )
    def gather_kernel(table_hbm, idx_hbm, out_hbm, idx_v, rows_v, sem):
        wid = lax.axis_index("s") * NC + lax.axis_index("c")
        base = wid * b_per_w
        pltpu.sync_copy(
            idx_hbm.at[pl.ds(wid * n_chunks, n_chunks), :], idx_v
        )
        copies = []
        for j in range(n_chunks):
            copies.append(
                pltpu.make_async_copy(
                    table_hbm.at[idx_v.at[j]],
                    rows_v.at[pl.ds(j * 128, 128), :],
                    sem,
                )
            )
            copies[-1].start()
        for c in copies:
            c.wait()
        pltpu.sync_copy(rows_v, out_hbm.at[pl.ds(base, b_per_w), :])

    return gather_kernel


def kernel(entities, entity_table, relation_table):
    B = entities.shape[0]
    V, D = entity_table.shape
    tt = entity_table.T
    table2 = _make_transpose(D, V)(tt, tt)
    idx = entities.astype(jnp.int32)
    lb = _BN.bit_length() - 1
    hi = (idx >> lb) & 1
    k = ((idx >> (lb + 1)) << lb) | (idx & (_BN - 1))
    gather = _make_pair_gather(table2.shape[0], 2 * D, B)
    pairs = gather(table2, k.reshape(B // 128, 128))
    return jnp.where((hi == 1)[:, None], pairs[:, D:], pairs[:, :D])
